# Initial kernel scaffold; baseline (speedup 1.0000x reference)
import jax
import jax.numpy as jnp
from jax.experimental import pallas as pl


def kernel(x):
    raise NotImplementedError("write your pallas kernel here")


# speedup vs baseline: 20.7106x; 20.7106x over previous
import jax
import jax.numpy as jnp
from jax.experimental import pallas as pl

_ROWS = 16384
_COLS = 200
_BLOCK_ROWS = 2048


def _perturb_body(x_ref, xc_ref, y_ref):
    b = x_ref[...]
    xc_ref[...] = b
    y_ref[...] = b
    y_ref[:, 0:1] = b[:, 1:2]
    y_ref[:, 1:2] = b[:, 0:1]


def kernel(x):
    rows, cols = x.shape
    block = min(_BLOCK_ROWS, rows)
    grid = (rows // block,)
    spec = pl.BlockSpec((block, cols), lambda i: (i, 0))
    xc, y = pl.pallas_call(
        _perturb_body,
        grid=grid,
        in_specs=[spec],
        out_specs=[spec, spec],
        out_shape=[
            jax.ShapeDtypeStruct((rows, cols), x.dtype),
            jax.ShapeDtypeStruct((rows, cols), x.dtype),
        ],
    )(x)
    return (xc, y, xc, y, xc)
